# baseline (speedup 1.0000x reference)
import functools

import jax
import jax.numpy as jnp
from jax import lax
from jax.experimental import pallas as pl
from jax.experimental.pallas import tpu as pltpu
from jax.experimental.pallas import tpu_sc as plsc

B = 4096
EMB = 64
L_DIAG = 50
L_DRUG = 20
H1, H2 = 512, 256

_NW = 32
_CHUNK = 128
_NBUF = 4
_NDEMO = 4
_NREP = 32
_DREP = 32
_BB = 512
_NB = B // _BB

_SEGS = (L_DIAG, L_DRUG, _NDEMO)
_K = tuple(s * EMB for s in _SEGS)


def _sc_gather(icd, drug, demo, ids_diag, ids_drug, ids_demo):
    mesh = plsc.VectorSubcoreMesh(core_axis_name="c", subcore_axis_name="s")
    cpws = tuple(B * s // _CHUNK // _NW for s in _SEGS)

    @functools.partial(
        pl.kernel,
        mesh=mesh,
        compiler_params=pltpu.CompilerParams(use_tc_tiling_on_sc=False),
        out_type=[jax.ShapeDtypeStruct((B * s, EMB), jnp.bfloat16)
                  for s in _SEGS],
        scratch_types=[
            pltpu.VMEM((sum(cpws) * _CHUNK,), jnp.int32),
            [pltpu.VMEM((_CHUNK, EMB), jnp.bfloat16)] * _NBUF,
            [pltpu.SemaphoreType.DMA] * _NBUF,
            [pltpu.SemaphoreType.DMA] * _NBUF,
        ],
    )
    def gather_k(icd_hbm, drug_hbm, demo_hbm, idd_hbm, idr_hbm, idm_hbm,
                 od_hbm, or_hbm, om_hbm, idx_v, bufs, gsems, wsems):
        wid = lax.axis_index("s") * 2 + lax.axis_index("c")

        def seg(table_hbm, ids_hbm, out_hbm, cpw, idx_base):
            ipw = cpw * _CHUNK
            c0 = wid * cpw
            pltpu.sync_copy(ids_hbm.at[pl.ds(wid * ipw, ipw)],
                            idx_v.at[pl.ds(idx_base, ipw)])

            def src(j):
                return table_hbm.at[
                    idx_v.at[pl.ds(idx_base + j * _CHUNK, _CHUNK)]]

            def dst(j):
                return out_hbm.at[pl.ds((c0 + j) * _CHUNK, _CHUNK)]

            ngrp = cpw // _NBUF
            tail = ngrp * _NBUF

            def group(g, _):
                base = g * _NBUF
                for k in range(_NBUF):
                    @pl.when(g > 0)
                    def _():
                        pltpu.make_async_copy(bufs[k], dst(0), wsems[k]).wait()
                    pltpu.async_copy(src(base + k), bufs[k], gsems[k])
                for k in range(_NBUF):
                    pltpu.make_async_copy(src(base + k), bufs[k],
                                          gsems[k]).wait()
                    pltpu.async_copy(bufs[k], dst(base + k), wsems[k])
                return 0

            lax.fori_loop(0, ngrp, group, 0)
            for k, j in enumerate(range(tail, cpw)):
                @pl.when(ngrp > 0)
                def _():
                    pltpu.make_async_copy(bufs[k], dst(0), wsems[k]).wait()
                pltpu.async_copy(src(j), bufs[k], gsems[k]).wait()
                pltpu.async_copy(bufs[k], dst(j), wsems[k])
            for k in range(min(_NBUF, cpw)):
                pltpu.make_async_copy(bufs[k], dst(0), wsems[k]).wait()

        seg(icd_hbm, idd_hbm, od_hbm, cpws[0], 0)
        seg(drug_hbm, idr_hbm, or_hbm, cpws[1], cpws[0] * _CHUNK)
        seg(demo_hbm, idm_hbm, om_hbm, cpws[2], (cpws[0] + cpws[1]) * _CHUNK)

    return gather_k(icd, drug, demo, ids_diag, ids_drug, ids_demo)


def _mlp_body(xd_ref, xi_ref, xr_ref, w1d_ref, w1i_ref, w1r_ref,
              b1_ref, g1_ref, bt1_ref, w2_ref, b2_ref,
              g2_ref, bt2_ref, w3_ref, b3_ref, out_ref,
              h1_ref, s1_ref, s2_ref):
    i = pl.program_id(0)
    h = jnp.dot(xi_ref[...].astype(jnp.bfloat16), w1i_ref[...],
                preferred_element_type=jnp.float32)
    h += jnp.dot(xr_ref[...].astype(jnp.bfloat16), w1r_ref[...],
                 preferred_element_type=jnp.float32)
    h += jnp.dot(xd_ref[...].astype(jnp.bfloat16), w1d_ref[...],
                 preferred_element_type=jnp.float32)
    h = jnp.maximum(h + b1_ref[...], 0.0)
    h1_ref[pl.ds(i * _BB, _BB), :] = h
    colsum = jnp.sum(h, axis=0, keepdims=True)
    colsq = jnp.sum(h * h, axis=0, keepdims=True)

    @pl.when(i == 0)
    def _():
        s1_ref[...] = colsum
        s2_ref[...] = colsq

    @pl.when(i > 0)
    def _():
        s1_ref[...] = s1_ref[...] + colsum
        s2_ref[...] = s2_ref[...] + colsq

    @pl.when(i == _NB - 1)
    def _():
        inv_b = 1.0 / B
        mean = s1_ref[...] * inv_b
        var = s2_ref[...] * inv_b - mean * mean
        scale = g1_ref[...] * lax.rsqrt(var + 1e-5)
        shift = bt1_ref[...] - mean * scale
        h1n = h1_ref[...] * scale + shift
        h2 = jnp.dot(h1n, w2_ref[...], preferred_element_type=jnp.float32)
        h2 = jnp.maximum(h2 + b2_ref[...], 0.0)
        m2 = jnp.sum(h2, axis=0, keepdims=True) * inv_b
        v2 = jnp.sum(h2 * h2, axis=0, keepdims=True) * inv_b - m2 * m2
        sc2 = g2_ref[...] * lax.rsqrt(v2 + 1e-5)
        sh2 = bt2_ref[...] - m2 * sc2
        h2n = h2 * sc2 + sh2
        out_ref[...] = (jnp.sum(h2n * w3_ref[...], axis=1, keepdims=True)
                        + b3_ref[...])


def _mlp(xd, xi, xr, W1d, W1i, W1r, b1, g1, beta1, W2, b2, g2, beta2, W3, b3):
    full = lambda shape: pl.BlockSpec(shape, lambda i: (0, 0))
    return pl.pallas_call(
        _mlp_body,
        grid=(_NB,),
        in_specs=[
            pl.BlockSpec((_BB, _K[2]), lambda i: (i, 0)),
            pl.BlockSpec((_BB, _K[0]), lambda i: (i, 0)),
            pl.BlockSpec((_BB, _K[1]), lambda i: (i, 0)),
            full((_K[2], H1)), full((_K[0], H1)), full((_K[1], H1)),
            full((1, H1)), full((1, H1)), full((1, H1)),
            full((H1, H2)),
            full((1, H2)), full((1, H2)), full((1, H2)),
            full((1, H2)),
            full((1, 1)),
        ],
        out_specs=full((B, 1)),
        out_shape=jax.ShapeDtypeStruct((B, 1), jnp.float32),
        scratch_shapes=[
            pltpu.VMEM((B, H1), jnp.float32),
            pltpu.VMEM((1, H1), jnp.float32),
            pltpu.VMEM((1, H1), jnp.float32),
        ],
    )(xd, xi, xr, W1d, W1i, W1r,
      b1.reshape(1, H1), g1.reshape(1, H1), beta1.reshape(1, H1),
      W2, b2.reshape(1, H2), g2.reshape(1, H2), beta2.reshape(1, H2),
      W3.reshape(1, H2), b3.reshape(1, 1))


def kernel(diag_ids, drug_ids, age_ids, race_ids, gender_ids,
           icd_emb, drug_emb, age_emb, race_emb, gender_emb,
           W1, b1, g1, beta1, W2, b2, g2, beta2, W3, b3):
    demo_rep = jnp.tile(
        jnp.concatenate([age_emb, race_emb, gender_emb,
                         jnp.zeros((1, EMB), jnp.float32)], axis=0),
        (_NREP, 1))
    rep = (jnp.arange(B, dtype=jnp.int32) % _NREP) * _DREP
    ids_demo = jnp.stack(
        [age_ids.astype(jnp.int32) + rep,
         race_ids.astype(jnp.int32) + rep + 20,
         gender_ids.astype(jnp.int32) + rep + 28,
         rep + 31], axis=1).reshape(B * _NDEMO)
    ids_diag = diag_ids.astype(jnp.int32).reshape(B * L_DIAG)
    ids_drug = drug_ids.astype(jnp.int32).reshape(B * L_DRUG)

    xi, xr, xd = _sc_gather(icd_emb.astype(jnp.bfloat16),
                            drug_emb.astype(jnp.bfloat16),
                            demo_rep.astype(jnp.bfloat16),
                            ids_diag, ids_drug, ids_demo)
    xi = xi.reshape(B, _K[0])
    xr = xr.reshape(B, _K[1])
    xd = xd.reshape(B, _K[2])

    w1 = W1.astype(jnp.bfloat16)
    W1d = jnp.concatenate(
        [w1[:3 * EMB], jnp.zeros((EMB, H1), jnp.bfloat16)], axis=0)
    W1i = w1[3 * EMB:(3 + L_DIAG) * EMB]
    W1r = w1[(3 + L_DIAG) * EMB:]
    out = _mlp(xd, xi, xr, W1d, W1i, W1r,
               b1, g1, beta1, W2, b2, g2, beta2, W3, b3)
    return out[:, 0]


# speedup vs baseline: 1.2861x; 1.2861x over previous
import functools

import jax
import jax.numpy as jnp
from jax import lax
from jax.experimental import pallas as pl
from jax.experimental.pallas import tpu as pltpu
from jax.experimental.pallas import tpu_sc as plsc

B = 4096
EMB = 64
L_DIAG = 50
L_DRUG = 20
H1, H2 = 512, 256

_NW = 32
_CHUNK = 128
_NBUF = 8
_NDEMO = 4
_NREP = 32
_DREP = 32
_BB = 512
_NB = B // _BB

_SEGS = (L_DIAG, L_DRUG, _NDEMO)
_K = tuple(s * EMB for s in _SEGS)


def _sc_gather(icd, drug, demo, ids_diag, ids_drug, ids_demo):
    mesh = plsc.VectorSubcoreMesh(core_axis_name="c", subcore_axis_name="s")
    cpws = tuple(B * s // _CHUNK // _NW for s in _SEGS)

    @functools.partial(
        pl.kernel,
        mesh=mesh,
        compiler_params=pltpu.CompilerParams(use_tc_tiling_on_sc=False),
        out_type=[jax.ShapeDtypeStruct((B * s, EMB), jnp.float32)
                  for s in _SEGS],
        scratch_types=[
            pltpu.VMEM((sum(cpws) * _CHUNK,), jnp.int32),
            [pltpu.VMEM((_CHUNK, EMB), jnp.float32)] * _NBUF,
            [pltpu.SemaphoreType.DMA] * _NBUF,
            [pltpu.SemaphoreType.DMA] * _NBUF,
        ],
    )
    def gather_k(icd_hbm, drug_hbm, demo_hbm, idd_hbm, idr_hbm, idm_hbm,
                 od_hbm, or_hbm, om_hbm, idx_v, bufs, gsems, wsems):
        wid = lax.axis_index("s") * 2 + lax.axis_index("c")

        def seg(table_hbm, ids_hbm, out_hbm, cpw, idx_base):
            ipw = cpw * _CHUNK
            c0 = wid * cpw
            pltpu.sync_copy(ids_hbm.at[pl.ds(wid * ipw, ipw)],
                            idx_v.at[pl.ds(idx_base, ipw)])

            def src(j):
                return table_hbm.at[
                    idx_v.at[pl.ds(idx_base + j * _CHUNK, _CHUNK)]]

            def dst(j):
                return out_hbm.at[pl.ds((c0 + j) * _CHUNK, _CHUNK)]

            ngrp = cpw // _NBUF
            tail = ngrp * _NBUF

            def group(g, _):
                base = g * _NBUF
                for k in range(_NBUF):
                    @pl.when(g > 0)
                    def _():
                        pltpu.make_async_copy(bufs[k], dst(0), wsems[k]).wait()
                    pltpu.async_copy(src(base + k), bufs[k], gsems[k])
                for k in range(_NBUF):
                    pltpu.make_async_copy(src(base + k), bufs[k],
                                          gsems[k]).wait()
                    pltpu.async_copy(bufs[k], dst(base + k), wsems[k])
                return 0

            lax.fori_loop(0, ngrp, group, 0)
            for k, j in enumerate(range(tail, cpw)):
                @pl.when(ngrp > 0)
                def _():
                    pltpu.make_async_copy(bufs[k], dst(0), wsems[k]).wait()
                pltpu.async_copy(src(j), bufs[k], gsems[k]).wait()
                pltpu.async_copy(bufs[k], dst(j), wsems[k])
            for k in range(min(_NBUF, cpw)):
                pltpu.make_async_copy(bufs[k], dst(0), wsems[k]).wait()

        seg(icd_hbm, idd_hbm, od_hbm, cpws[0], 0)
        seg(drug_hbm, idr_hbm, or_hbm, cpws[1], cpws[0] * _CHUNK)
        seg(demo_hbm, idm_hbm, om_hbm, cpws[2], (cpws[0] + cpws[1]) * _CHUNK)

    return gather_k(icd, drug, demo, ids_diag, ids_drug, ids_demo)


def _mlp_body(xd_ref, xi_ref, xr_ref, w1d_ref, w1i_ref, w1r_ref,
              b1_ref, g1_ref, bt1_ref, w2_ref, b2_ref,
              g2_ref, bt2_ref, w3_ref, b3_ref, out_ref,
              h1_ref, s1_ref, s2_ref):
    i = pl.program_id(0)
    h = jnp.dot(xi_ref[...].astype(jnp.bfloat16), w1i_ref[...],
                preferred_element_type=jnp.float32)
    h += jnp.dot(xr_ref[...].astype(jnp.bfloat16), w1r_ref[...],
                 preferred_element_type=jnp.float32)
    h += jnp.dot(xd_ref[...].astype(jnp.bfloat16), w1d_ref[...],
                 preferred_element_type=jnp.float32)
    h = jnp.maximum(h + b1_ref[...], 0.0)
    h1_ref[pl.ds(i * _BB, _BB), :] = h
    colsum = jnp.sum(h, axis=0, keepdims=True)
    colsq = jnp.sum(h * h, axis=0, keepdims=True)

    @pl.when(i == 0)
    def _():
        s1_ref[...] = colsum
        s2_ref[...] = colsq

    @pl.when(i > 0)
    def _():
        s1_ref[...] = s1_ref[...] + colsum
        s2_ref[...] = s2_ref[...] + colsq

    @pl.when(i == _NB - 1)
    def _():
        inv_b = 1.0 / B
        mean = s1_ref[...] * inv_b
        var = s2_ref[...] * inv_b - mean * mean
        scale = g1_ref[...] * lax.rsqrt(var + 1e-5)
        shift = bt1_ref[...] - mean * scale
        h1n = h1_ref[...] * scale + shift
        h2 = jnp.dot(h1n, w2_ref[...], preferred_element_type=jnp.float32)
        h2 = jnp.maximum(h2 + b2_ref[...], 0.0)
        m2 = jnp.sum(h2, axis=0, keepdims=True) * inv_b
        v2 = jnp.sum(h2 * h2, axis=0, keepdims=True) * inv_b - m2 * m2
        sc2 = g2_ref[...] * lax.rsqrt(v2 + 1e-5)
        sh2 = bt2_ref[...] - m2 * sc2
        h2n = h2 * sc2 + sh2
        out_ref[...] = (jnp.sum(h2n * w3_ref[...], axis=1, keepdims=True)
                        + b3_ref[...])


def _mlp(xd, xi, xr, W1d, W1i, W1r, b1, g1, beta1, W2, b2, g2, beta2, W3, b3):
    full = lambda shape: pl.BlockSpec(shape, lambda i: (0, 0))
    return pl.pallas_call(
        _mlp_body,
        grid=(_NB,),
        in_specs=[
            pl.BlockSpec((_BB, _K[2]), lambda i: (i, 0)),
            pl.BlockSpec((_BB, _K[0]), lambda i: (i, 0)),
            pl.BlockSpec((_BB, _K[1]), lambda i: (i, 0)),
            full((_K[2], H1)), full((_K[0], H1)), full((_K[1], H1)),
            full((1, H1)), full((1, H1)), full((1, H1)),
            full((H1, H2)),
            full((1, H2)), full((1, H2)), full((1, H2)),
            full((1, H2)),
            full((1, 1)),
        ],
        out_specs=full((B, 1)),
        out_shape=jax.ShapeDtypeStruct((B, 1), jnp.float32),
        scratch_shapes=[
            pltpu.VMEM((B, H1), jnp.float32),
            pltpu.VMEM((1, H1), jnp.float32),
            pltpu.VMEM((1, H1), jnp.float32),
        ],
    )(xd, xi, xr, W1d, W1i, W1r,
      b1.reshape(1, H1), g1.reshape(1, H1), beta1.reshape(1, H1),
      W2, b2.reshape(1, H2), g2.reshape(1, H2), beta2.reshape(1, H2),
      W3.reshape(1, H2), b3.reshape(1, 1))


def kernel(diag_ids, drug_ids, age_ids, race_ids, gender_ids,
           icd_emb, drug_emb, age_emb, race_emb, gender_emb,
           W1, b1, g1, beta1, W2, b2, g2, beta2, W3, b3):
    demo_rep = jnp.tile(
        jnp.concatenate([age_emb, race_emb, gender_emb,
                         jnp.zeros((1, EMB), jnp.float32)], axis=0),
        (_NREP, 1))
    rep = (jnp.arange(B, dtype=jnp.int32) % _NREP) * _DREP
    ids_demo = jnp.stack(
        [age_ids.astype(jnp.int32) + rep,
         race_ids.astype(jnp.int32) + rep + 20,
         gender_ids.astype(jnp.int32) + rep + 28,
         rep + 31], axis=1).reshape(B * _NDEMO)
    ids_diag = diag_ids.astype(jnp.int32).reshape(B * L_DIAG)
    ids_drug = drug_ids.astype(jnp.int32).reshape(B * L_DRUG)

    xi, xr, xd = _sc_gather(icd_emb, drug_emb, demo_rep,
                            ids_diag, ids_drug, ids_demo)
    xi = xi.reshape(B, _K[0])
    xr = xr.reshape(B, _K[1])
    xd = xd.reshape(B, _K[2])

    w1 = W1.astype(jnp.bfloat16)
    W1d = jnp.concatenate(
        [w1[:3 * EMB], jnp.zeros((EMB, H1), jnp.bfloat16)], axis=0)
    W1i = w1[3 * EMB:(3 + L_DIAG) * EMB]
    W1r = w1[(3 + L_DIAG) * EMB:]
    out = _mlp(xd, xi, xr, W1d, W1i, W1r,
               b1, g1, beta1, W2, b2, g2, beta2, W3, b3)
    return out[:, 0]


# speedup vs baseline: 1.3104x; 1.0189x over previous
import functools

import jax
import jax.numpy as jnp
from jax import lax
from jax.experimental import pallas as pl
from jax.experimental.pallas import tpu as pltpu
from jax.experimental.pallas import tpu_sc as plsc

B = 4096
EMB = 64
L_DIAG = 50
L_DRUG = 20
H1, H2 = 512, 256

_NW = 32
_CHUNK = 128
_NBUF = 4
_NDEMO = 4
_NREP = 32
_DREP = 32
_BB = 512
_NB = B // _BB

_SEGS = (L_DIAG, L_DRUG, _NDEMO)
_K = tuple(s * EMB for s in _SEGS)


def _worker_seg(wid, idx_v, bufs, gsems, wsems):
    def seg(table_hbm, ids_hbm, out_hbm, cpw, idx_base):
        ipw = cpw * _CHUNK
        c0 = wid * cpw
        pltpu.sync_copy(ids_hbm.at[pl.ds(wid * ipw, ipw)],
                        idx_v.at[pl.ds(idx_base, ipw)])

        def src(j):
            return table_hbm.at[
                idx_v.at[pl.ds(idx_base + j * _CHUNK, _CHUNK)]]

        def dst(j):
            return out_hbm.at[pl.ds((c0 + j) * _CHUNK, _CHUNK)]

        ngrp = cpw // _NBUF
        tail = ngrp * _NBUF

        def group(g, _):
            base = g * _NBUF
            for k in range(_NBUF):
                @pl.when(g > 0)
                def _():
                    pltpu.make_async_copy(bufs[k], dst(0), wsems[k]).wait()
                pltpu.async_copy(src(base + k), bufs[k], gsems[k])
            for k in range(_NBUF):
                pltpu.make_async_copy(src(base + k), bufs[k],
                                      gsems[k]).wait()
                pltpu.async_copy(bufs[k], dst(base + k), wsems[k])
            return 0

        lax.fori_loop(0, ngrp, group, 0)
        for k, j in enumerate(range(tail, cpw)):
            @pl.when(ngrp > 0)
            def _():
                pltpu.make_async_copy(bufs[k], dst(0), wsems[k]).wait()
            pltpu.async_copy(src(j), bufs[k], gsems[k]).wait()
            pltpu.async_copy(bufs[k], dst(j), wsems[k])
        for k in range(min(_NBUF, cpw)):
            pltpu.make_async_copy(bufs[k], dst(0), wsems[k]).wait()

    return seg


def _sc_gather_diag(icd, ids_diag):
    mesh = plsc.VectorSubcoreMesh(core_axis_name="c", subcore_axis_name="s")
    cpw = B * L_DIAG // _CHUNK // _NW

    @functools.partial(
        pl.kernel,
        mesh=mesh,
        compiler_params=pltpu.CompilerParams(use_tc_tiling_on_sc=False),
        out_type=jax.ShapeDtypeStruct((B * L_DIAG, EMB), jnp.float32),
        scratch_types=[
            pltpu.VMEM((cpw * _CHUNK,), jnp.int32),
            [pltpu.VMEM((_CHUNK, EMB), jnp.float32)] * _NBUF,
            [pltpu.SemaphoreType.DMA] * _NBUF,
            [pltpu.SemaphoreType.DMA] * _NBUF,
        ],
    )
    def gather_k(icd_hbm, idd_hbm, od_hbm, idx_v, bufs, gsems, wsems):
        wid = lax.axis_index("s") * 2 + lax.axis_index("c")
        seg = _worker_seg(wid, idx_v, bufs, gsems, wsems)
        seg(icd_hbm, idd_hbm, od_hbm, cpw, 0)

    return gather_k(icd, ids_diag)


def _sc_gather_rest(drug, demo, ids_drug, ids_demo):
    mesh = plsc.VectorSubcoreMesh(core_axis_name="c", subcore_axis_name="s")
    cpws = (B * L_DRUG // _CHUNK // _NW, B * _NDEMO // _CHUNK // _NW)

    @functools.partial(
        pl.kernel,
        mesh=mesh,
        compiler_params=pltpu.CompilerParams(use_tc_tiling_on_sc=False),
        out_type=[jax.ShapeDtypeStruct((B * L_DRUG, EMB), jnp.float32),
                  jax.ShapeDtypeStruct((B * _NDEMO, EMB), jnp.float32)],
        scratch_types=[
            pltpu.VMEM((sum(cpws) * _CHUNK,), jnp.int32),
            [pltpu.VMEM((_CHUNK, EMB), jnp.float32)] * _NBUF,
            [pltpu.SemaphoreType.DMA] * _NBUF,
            [pltpu.SemaphoreType.DMA] * _NBUF,
        ],
    )
    def gather_k(drug_hbm, demo_hbm, idr_hbm, idm_hbm,
                 or_hbm, om_hbm, idx_v, bufs, gsems, wsems):
        wid = lax.axis_index("s") * 2 + lax.axis_index("c")
        seg = _worker_seg(wid, idx_v, bufs, gsems, wsems)
        seg(drug_hbm, idr_hbm, or_hbm, cpws[0], 0)
        seg(demo_hbm, idm_hbm, om_hbm, cpws[1], cpws[0] * _CHUNK)

    return gather_k(drug, demo, ids_drug, ids_demo)


def _mlp_body(xd_ref, xi_ref, xr_ref, w1d_ref, w1i_ref, w1r_ref,
              b1_ref, g1_ref, bt1_ref, w2_ref, b2_ref,
              g2_ref, bt2_ref, w3_ref, b3_ref, out_ref,
              h1_ref, s1_ref, s2_ref):
    i = pl.program_id(0)
    h = jnp.dot(xi_ref[...].astype(jnp.bfloat16), w1i_ref[...],
                preferred_element_type=jnp.float32)
    h += jnp.dot(xr_ref[...].astype(jnp.bfloat16), w1r_ref[...],
                 preferred_element_type=jnp.float32)
    h += jnp.dot(xd_ref[...].astype(jnp.bfloat16), w1d_ref[...],
                 preferred_element_type=jnp.float32)
    h = jnp.maximum(h + b1_ref[...], 0.0)
    h1_ref[pl.ds(i * _BB, _BB), :] = h
    colsum = jnp.sum(h, axis=0, keepdims=True)
    colsq = jnp.sum(h * h, axis=0, keepdims=True)

    @pl.when(i == 0)
    def _():
        s1_ref[...] = colsum
        s2_ref[...] = colsq

    @pl.when(i > 0)
    def _():
        s1_ref[...] = s1_ref[...] + colsum
        s2_ref[...] = s2_ref[...] + colsq

    @pl.when(i == _NB - 1)
    def _():
        inv_b = 1.0 / B
        mean = s1_ref[...] * inv_b
        var = s2_ref[...] * inv_b - mean * mean
        scale = g1_ref[...] * lax.rsqrt(var + 1e-5)
        shift = bt1_ref[...] - mean * scale
        h1n = h1_ref[...] * scale + shift
        h2 = jnp.dot(h1n, w2_ref[...], preferred_element_type=jnp.float32)
        h2 = jnp.maximum(h2 + b2_ref[...], 0.0)
        m2 = jnp.sum(h2, axis=0, keepdims=True) * inv_b
        v2 = jnp.sum(h2 * h2, axis=0, keepdims=True) * inv_b - m2 * m2
        sc2 = g2_ref[...] * lax.rsqrt(v2 + 1e-5)
        sh2 = bt2_ref[...] - m2 * sc2
        h2n = h2 * sc2 + sh2
        out_ref[...] = (jnp.sum(h2n * w3_ref[...], axis=1, keepdims=True)
                        + b3_ref[...])


def _mlp(xd, xi, xr, W1d, W1i, W1r, b1, g1, beta1, W2, b2, g2, beta2, W3, b3):
    full = lambda shape: pl.BlockSpec(shape, lambda i: (0, 0))
    return pl.pallas_call(
        _mlp_body,
        grid=(_NB,),
        in_specs=[
            pl.BlockSpec((_BB, _K[2]), lambda i: (i, 0)),
            pl.BlockSpec((_BB, _K[0]), lambda i: (i, 0)),
            pl.BlockSpec((_BB, _K[1]), lambda i: (i, 0)),
            full((_K[2], H1)), full((_K[0], H1)), full((_K[1], H1)),
            full((1, H1)), full((1, H1)), full((1, H1)),
            full((H1, H2)),
            full((1, H2)), full((1, H2)), full((1, H2)),
            full((1, H2)),
            full((1, 1)),
        ],
        out_specs=full((B, 1)),
        out_shape=jax.ShapeDtypeStruct((B, 1), jnp.float32),
        scratch_shapes=[
            pltpu.VMEM((B, H1), jnp.float32),
            pltpu.VMEM((1, H1), jnp.float32),
            pltpu.VMEM((1, H1), jnp.float32),
        ],
    )(xd, xi, xr, W1d, W1i, W1r,
      b1.reshape(1, H1), g1.reshape(1, H1), beta1.reshape(1, H1),
      W2, b2.reshape(1, H2), g2.reshape(1, H2), beta2.reshape(1, H2),
      W3.reshape(1, H2), b3.reshape(1, 1))


def kernel(diag_ids, drug_ids, age_ids, race_ids, gender_ids,
           icd_emb, drug_emb, age_emb, race_emb, gender_emb,
           W1, b1, g1, beta1, W2, b2, g2, beta2, W3, b3):
    demo_rep = jnp.tile(
        jnp.concatenate([age_emb, race_emb, gender_emb,
                         jnp.zeros((1, EMB), jnp.float32)], axis=0),
        (_NREP, 1))
    rep = (jnp.arange(B, dtype=jnp.int32) % _NREP) * _DREP
    ids_demo = jnp.stack(
        [age_ids.astype(jnp.int32) + rep,
         race_ids.astype(jnp.int32) + rep + 20,
         gender_ids.astype(jnp.int32) + rep + 28,
         rep + 31], axis=1).reshape(B * _NDEMO)
    ids_diag = diag_ids.astype(jnp.int32).reshape(B * L_DIAG)
    ids_drug = drug_ids.astype(jnp.int32).reshape(B * L_DRUG)

    xi = _sc_gather_diag(icd_emb, ids_diag)
    xr, xd = _sc_gather_rest(drug_emb, demo_rep, ids_drug, ids_demo)
    xi = xi.reshape(B, _K[0])
    xr = xr.reshape(B, _K[1])
    xd = xd.reshape(B, _K[2])

    w1 = W1.astype(jnp.bfloat16)
    W1d = jnp.concatenate(
        [w1[:3 * EMB], jnp.zeros((EMB, H1), jnp.bfloat16)], axis=0)
    W1i = w1[3 * EMB:(3 + L_DIAG) * EMB]
    W1r = w1[(3 + L_DIAG) * EMB:]
    out = _mlp(xd, xi, xr, W1d, W1i, W1r,
               b1, g1, beta1, W2, b2, g2, beta2, W3, b3)
    return out[:, 0]
